# Initial kernel scaffold; baseline (speedup 1.0000x reference)
import jax
import jax.numpy as jnp
from jax.experimental import pallas as pl


def kernel(x, edge_index, W1l, b1, W1r, W2l, b2, W2r, W_ih, W_hh, b_ih, b_hh, W3, b3, W4, b4):
    raise NotImplementedError("write your pallas kernel here")


# speedup vs baseline: 7.3975x; 7.3975x over previous
import functools

import jax
import jax.numpy as jnp
from jax import lax
from jax.experimental import pallas as pl
from jax.experimental.pallas import tpu as pltpu
from jax.experimental.pallas import tpu_sc as plsc

TT, NN, EE = 8, 50000, 800000
IN_DIM, HID = 15, 64

NCORE, NSUB = 2, 16
NWORK = NCORE * NSUB
N_PAD = 51200
E_PAD = 819200
W_EDGES = E_PAD // NWORK
CH = 2560
NCHUNK = W_EDGES // CH
ROWS_A = N_PAD // NSUB
ZROWS = 800
N_HALF = N_PAD // NCORE
TRASH = N_HALF
ACC_ROWS = N_HALF + 8
SUB_ROWS = N_HALF // NSUB
W_A = E_PAD // NSUB
NCHUNK_A = W_A // CH

_MESH = plsc.VectorSubcoreMesh(
    core_axis_name="c", subcore_axis_name="s", num_cores=NCORE)


@functools.partial(
    pl.kernel,
    out_type=jax.ShapeDtypeStruct((TT, N_PAD, 16), jnp.float32),
    mesh=_MESH,
    compiler_params=pltpu.CompilerParams(use_tc_tiling_on_sc=False),
    scratch_types=[
        pltpu.VMEM((CH,), jnp.int32),
        pltpu.VMEM((CH,), jnp.int32),
        pltpu.VMEM((CH, 16), jnp.float32),
        pltpu.VMEM((CH,), jnp.int32),
        pltpu.VMEM((CH,), jnp.int32),
        pltpu.VMEM((CH, 16), jnp.float32),
        pltpu.VMEM((ZROWS, 16), jnp.float32),
        pltpu.VMEM_SHARED((ACC_ROWS, 16), jnp.float32),
        pltpu.SemaphoreType.DMA,
        pltpu.SemaphoreType.DMA,
    ],
)
def _seg_sum(xflat, srcabs, dstm, out, src0, dst0, rows0, src1, dst1, rows1,
             zero_v, acc, sem0, sem1):
    c = lax.axis_index("c")
    s = lax.axis_index("s")
    lo = c * N_HALF

    def _zb(i, _):
        zero_v[i, :] = jnp.zeros((16,), jnp.float32)
        return _
    lax.fori_loop(0, ZROWS, _zb, None)

    def _remap(dst_v):
        def _rm(i, _):
            d = dst_v[pl.ds(i * 16, 16)] - lo
            ok = (d >= 0) & (d < N_HALF)
            dst_v[pl.ds(i * 16, 16)] = jnp.where(ok, d, TRASH)
            return _
        lax.fori_loop(0, CH // 16, _rm, None)

    def _tbody(t, _):
        for z in range(SUB_ROWS // ZROWS):
            pltpu.sync_copy(zero_v, acc.at[pl.ds(s * SUB_ROWS + z * ZROWS, ZROWS)])
        plsc.subcore_barrier()
        ebase = s * W_A

        pltpu.sync_copy(srcabs.at[t, pl.ds(ebase, CH)], src0)
        pltpu.sync_copy(dstm.at[t, pl.ds(ebase, CH)], dst0)
        pltpu.async_copy(xflat.at[src0], rows0, sem0)

        def _pair(j, _):
            b1 = ebase + (2 * j + 1) * CH
            pltpu.sync_copy(srcabs.at[t, pl.ds(b1, CH)], src1)
            pltpu.sync_copy(dstm.at[t, pl.ds(b1, CH)], dst1)
            pltpu.async_copy(xflat.at[src1], rows1, sem1)
            _remap(dst0)
            pltpu.make_async_copy(xflat.at[src0], rows0, sem0).wait()
            pltpu.sync_copy(rows0, acc.at[dst0], add=True)
            @pl.when(j + 1 < NCHUNK_A // 2)
            def _():
                b2 = ebase + (2 * j + 2) * CH
                pltpu.sync_copy(srcabs.at[t, pl.ds(b2, CH)], src0)
                pltpu.sync_copy(dstm.at[t, pl.ds(b2, CH)], dst0)
                pltpu.async_copy(xflat.at[src0], rows0, sem0)
            _remap(dst1)
            pltpu.make_async_copy(xflat.at[src1], rows1, sem1).wait()
            pltpu.sync_copy(rows1, acc.at[dst1], add=True)
            return _
        lax.fori_loop(0, NCHUNK_A // 2, _pair, None)
        plsc.subcore_barrier()
        pltpu.sync_copy(acc.at[pl.ds(s * SUB_ROWS, SUB_ROWS)],
                        out.at[t, pl.ds(c * N_HALF + s * SUB_ROWS, SUB_ROWS)])
        return _
    lax.fori_loop(0, TT, _tbody, None)


@functools.partial(
    pl.kernel,
    out_type=jax.ShapeDtypeStruct((TT, NCORE, N_PAD), jnp.float32),
    mesh=_MESH,
    compiler_params=pltpu.CompilerParams(use_tc_tiling_on_sc=False),
    scratch_types=[
        pltpu.VMEM((CH,), jnp.int32),
        pltpu.VMEM((CH,), jnp.int32),
        pltpu.VMEM((CH,), jnp.float32),
        pltpu.VMEM((CH,), jnp.int32),
        pltpu.VMEM((CH,), jnp.int32),
        pltpu.VMEM((CH,), jnp.float32),
        pltpu.VMEM((ROWS_A,), jnp.float32),
        pltpu.VMEM((ROWS_A,), jnp.float32),
        pltpu.VMEM_SHARED((N_PAD,), jnp.float32),
        pltpu.VMEM_SHARED((N_PAD,), jnp.float32),
        pltpu.SemaphoreType.DMA,
        pltpu.SemaphoreType.DMA,
    ],
)
def _q_pass(invcnt, srcp, dstm, out, src0, dst0, w0, src1, dst1, w1,
            stage_v, zero_v, inv_sh, q_sh, sem0, sem1):
    c = lax.axis_index("c")
    s = lax.axis_index("s")
    w = s * NCORE + c

    def _zb(i, _):
        zero_v[pl.ds(i * 16, 16)] = jnp.zeros((16,), jnp.float32)
        return _
    lax.fori_loop(0, ROWS_A // 16, _zb, None)

    def _tbody(t, _):
        pltpu.sync_copy(invcnt.at[t, 0, pl.ds(s * ROWS_A, ROWS_A)], stage_v)
        pltpu.sync_copy(stage_v, inv_sh.at[pl.ds(s * ROWS_A, ROWS_A)])
        pltpu.sync_copy(zero_v, q_sh.at[pl.ds(s * ROWS_A, ROWS_A)])
        plsc.subcore_barrier()
        ebase = w * W_EDGES

        pltpu.sync_copy(srcp.at[t, pl.ds(ebase, CH)], src0)
        pltpu.sync_copy(dstm.at[t, pl.ds(ebase, CH)], dst0)
        pltpu.async_copy(inv_sh.at[dst0], w0, sem0)

        def _pair(j, _):
            b1 = ebase + (2 * j + 1) * CH
            pltpu.sync_copy(srcp.at[t, pl.ds(b1, CH)], src1)
            pltpu.sync_copy(dstm.at[t, pl.ds(b1, CH)], dst1)
            pltpu.async_copy(inv_sh.at[dst1], w1, sem1)
            pltpu.make_async_copy(inv_sh.at[dst0], w0, sem0).wait()
            pltpu.sync_copy(w0, q_sh.at[src0], add=True)

            @pl.when(j + 1 < NCHUNK // 2)
            def _():
                b2 = ebase + (2 * j + 2) * CH
                pltpu.sync_copy(srcp.at[t, pl.ds(b2, CH)], src0)
                pltpu.sync_copy(dstm.at[t, pl.ds(b2, CH)], dst0)
                pltpu.async_copy(inv_sh.at[dst0], w0, sem0)

            pltpu.make_async_copy(inv_sh.at[dst1], w1, sem1).wait()
            pltpu.sync_copy(w1, q_sh.at[src1], add=True)
            return _
        lax.fori_loop(0, NCHUNK // 2, _pair, None)
        plsc.subcore_barrier()
        pltpu.sync_copy(q_sh.at[pl.ds(s * ROWS_A, ROWS_A)],
                        out.at[t, c, pl.ds(s * ROWS_A, ROWS_A)])
        return _
    lax.fori_loop(0, TT, _tbody, None)


NB_B = 2048
NBLK_B = N_PAD // NB_B


def _inv_body(part_ref, inv_ref):
    cnt = part_ref[...][:, :, 15]
    inv_ref[...] = (1.0 / jnp.maximum(cnt, 1.0))[:, None, :]


NB_D = 2048
NBLK_D = N_PAD // NB_D


def _dense_body(xp_ref, part_ref, inv_ref, q_ref, w1l_ref, b1_ref, w1r_ref,
                u_ref, v_ref):
    nb = pl.program_id(1)
    xb = xp_ref[0]
    s16 = part_ref[0]
    inv = inv_ref[0, 0][:, None]
    agg = s16 * inv
    h1 = agg @ w1l_ref[...] + xb @ w1r_ref[...] + b1_ref[...]
    h1 = jnp.maximum(h1, 0.0)
    row = nb * NB_D + lax.broadcasted_iota(jnp.int32, (NB_D, 1), 0)
    h1 = jnp.where(row < NN, h1, 0.0)
    qs = jnp.sum(q_ref[0], axis=0, keepdims=True)
    u_c = jnp.dot(qs, h1, preferred_element_type=jnp.float32)
    v_c = jnp.sum(h1, axis=0, keepdims=True)

    @pl.when(nb == 0)
    def _():
        u_ref[...] = jnp.zeros_like(u_ref)
        v_ref[...] = jnp.zeros_like(v_ref)

    u_ref[...] += u_c[None]
    v_ref[...] += v_c[None]


def _head_body(u_ref, v_ref, w2l_ref, b2_ref, w2r_ref, wih_ref, whh_ref,
               bih_ref, bhh_ref, w3_ref, b3_ref, w4_ref, b4_ref, out_ref):
    scale = 1.0 / NN
    emb = (u_ref[...] * scale) @ w2l_ref[...] + b2_ref[...] \
        + (v_ref[...] * scale) @ w2r_ref[...]
    gi_all = emb @ wih_ref[...] + bih_ref[...]
    h = jnp.zeros((1, HID), jnp.float32)
    for t in range(TT):
        gi = gi_all[t:t + 1, :]
        gh = h @ whh_ref[...] + bhh_ref[...]
        r = jax.nn.sigmoid(gi[:, :HID] + gh[:, :HID])
        z = jax.nn.sigmoid(gi[:, HID:2 * HID] + gh[:, HID:2 * HID])
        n_ = jnp.tanh(gi[:, 2 * HID:] + r * gh[:, 2 * HID:])
        h = (1.0 - z) * n_ + z * h
    hid = jnp.maximum(h @ w3_ref[...] + b3_ref[...], 0.0)
    out_ref[...] = hid @ w4_ref[...] + b4_ref[...]


def kernel(x, edge_index, W1l, b1, W1r, W2l, b2, W2r,
           W_ih, W_hh, b_ih, b_hh, W3, b3, W4, b4):
    f32 = jnp.float32
    xpad = jnp.zeros((TT, N_PAD, 16), f32)
    xpad = xpad.at[:, :NN, :IN_DIM].set(x)
    xpad = xpad.at[:, :NN, 15].set(1.0)
    pad_e = jnp.full((TT, E_PAD - EE), N_PAD - 1, jnp.int32)
    srcp = jnp.concatenate([edge_index[:, 0, :], pad_e], axis=1)
    dstm = jnp.concatenate([edge_index[:, 1, :], pad_e], axis=1)
    srcabs = srcp + (jnp.arange(TT, dtype=jnp.int32) * N_PAD)[:, None]
    xflat = xpad.reshape(TT * N_PAD, 16)

    part = _seg_sum(xflat, srcabs, dstm)

    invcnt = pl.pallas_call(
        _inv_body,
        grid=(TT, NBLK_B),
        in_specs=[pl.BlockSpec((1, NB_B, 16), lambda t, nb: (t, nb, 0))],
        out_specs=pl.BlockSpec((1, 1, NB_B), lambda t, nb: (t, 0, nb)),
        out_shape=jax.ShapeDtypeStruct((TT, 1, N_PAD), f32),
    )(part)

    qpart = _q_pass(invcnt, srcp, dstm)

    w1l_p = jnp.zeros((16, HID), f32).at[:IN_DIM].set(W1l.T)
    w1r_p = jnp.zeros((16, HID), f32).at[:IN_DIM].set(W1r.T)
    u, v = pl.pallas_call(
        _dense_body,
        grid=(TT, NBLK_D),
        in_specs=[
            pl.BlockSpec((1, NB_D, 16), lambda t, nb: (t, nb, 0)),
            pl.BlockSpec((1, NB_D, 16), lambda t, nb: (t, nb, 0)),
            pl.BlockSpec((1, 1, NB_D), lambda t, nb: (t, 0, nb)),
            pl.BlockSpec((1, NCORE, NB_D), lambda t, nb: (t, 0, nb)),
            pl.BlockSpec((16, HID), lambda t, nb: (0, 0)),
            pl.BlockSpec((1, HID), lambda t, nb: (0, 0)),
            pl.BlockSpec((16, HID), lambda t, nb: (0, 0)),
        ],
        out_specs=[
            pl.BlockSpec((1, 1, HID), lambda t, nb: (t, 0, 0)),
            pl.BlockSpec((1, 1, HID), lambda t, nb: (t, 0, 0)),
        ],
        out_shape=[
            jax.ShapeDtypeStruct((TT, 1, HID), f32),
            jax.ShapeDtypeStruct((TT, 1, HID), f32),
        ],
    )(xpad, part, invcnt, qpart, w1l_p, b1[None, :], w1r_p)
    u = u[:, 0, :]
    v = v[:, 0, :]

    w4_p = jnp.zeros((32, 8), f32).at[:, :3].set(W4.T)
    b4_p = jnp.zeros((1, 8), f32).at[0, :3].set(b4)
    logits8 = pl.pallas_call(
        _head_body,
        out_shape=jax.ShapeDtypeStruct((1, 8), f32),
    )(u, v, W2l.T, b2[None, :], W2r.T, W_ih.T, W_hh.T,
      b_ih[None, :], b_hh[None, :], W3.T, b3[None, :], w4_p, b4_p)
    return logits8[:, :3]
